# Initial kernel scaffold; baseline (speedup 1.0000x reference)
import jax
import jax.numpy as jnp
from jax.experimental import pallas as pl


def kernel(x, edge_index, batch, W1, b1, W2, b2, W3, b3, Wfc, bfc):
    raise NotImplementedError("write your pallas kernel here")


# speedup vs baseline: 8.0987x; 8.0987x over previous
import jax
import jax.numpy as jnp
from jax import lax
from jax.experimental import pallas as pl
from jax.experimental.pallas import tpu as pltpu
from jax.experimental.pallas import tpu_sc as plsc

N = 10000
NP = 10240
D = 128
E = 320000
G = 64
CLS = 10
NC, NS = 2, 16
NW = NC * NS
CH = 128
NCHUNK = -(-E // (NW * CH))
EP = NW * NCHUNK * CH
ROWS_PT = NP // NS
ZCH = ROWS_PT // CH
DW = 8

_mesh = plsc.VectorSubcoreMesh(core_axis_name="c", subcore_axis_name="s")


def _deg_body(dsts_hbm, ones_hbm, zeros_hbm, out_hbm, acc, dst_cur, ones_v):
    c = lax.axis_index("c")
    s = lax.axis_index("s")
    base = s * ROWS_PT
    pltpu.sync_copy(zeros_hbm, acc.at[pl.ds(base, ROWS_PT)])
    pltpu.sync_copy(ones_hbm, ones_v)
    plsc.subcore_barrier()

    def body(j, carry):
        pltpu.sync_copy(dsts_hbm.at[c, s, j], dst_cur)
        pltpu.sync_copy(ones_v, acc.at[dst_cur], add=True)
        return carry

    lax.fori_loop(0, NCHUNK, body, 0)
    plsc.subcore_barrier()
    pltpu.sync_copy(acc.at[pl.ds(base, ROWS_PT)], out_hbm.at[c, pl.ds(base, ROWS_PT)])


_deg = pl.kernel(
    _deg_body,
    out_type=jax.ShapeDtypeStruct((NC, NP, DW), jnp.float32),
    mesh=_mesh,
    scratch_types=[
        pltpu.VMEM_SHARED((NP, DW), jnp.float32),
        pltpu.VMEM((CH,), jnp.int32),
        pltpu.VMEM((CH, DW), jnp.float32),
    ],
)


def _agg_body(hp_hbm, srcs_hbm, dsts_hbm, zeros_hbm, out_hbm,
              acc, src_cur, dst_cur, rows_v, sem):
    c = lax.axis_index("c")
    s = lax.axis_index("s")
    base = s * ROWS_PT
    for k in range(ZCH):
        pltpu.sync_copy(zeros_hbm, acc.at[pl.ds(base + k * CH, CH)])
    plsc.subcore_barrier()

    def body(j, carry):
        pltpu.sync_copy(srcs_hbm.at[c, s, j], src_cur)
        pltpu.sync_copy(dsts_hbm.at[c, s, j], dst_cur)
        pltpu.async_copy(hp_hbm.at[src_cur], rows_v, sem).wait()
        pltpu.sync_copy(rows_v, acc.at[dst_cur], add=True)
        return carry

    lax.fori_loop(0, NCHUNK, body, 0)
    plsc.subcore_barrier()
    pltpu.sync_copy(acc.at[pl.ds(base, ROWS_PT)], out_hbm.at[c, pl.ds(base, ROWS_PT)])


_agg = pl.kernel(
    _agg_body,
    out_type=jax.ShapeDtypeStruct((NC, NP, D), jnp.float32),
    mesh=_mesh,
    scratch_types=[
        pltpu.VMEM_SHARED((NP, D), jnp.float32),
        pltpu.VMEM((CH,), jnp.int32),
        pltpu.VMEM((CH,), jnp.int32),
        pltpu.VMEM((CH, D), jnp.float32),
        pltpu.SemaphoreType.DMA,
    ],
)


def _t1_body(degp_ref, x_ref, w_ref, hp_ref, dinv_ref):
    deg = degp_ref[0, :, 0:1] + degp_ref[1, :, 0:1]
    dinv = lax.rsqrt(deg + 1.0)
    dinv_ref[...] = dinv
    hp_ref[...] = dinv * jnp.dot(x_ref[...], w_ref[...],
                                 preferred_element_type=jnp.float32)


_t1 = pl.pallas_call(
    _t1_body,
    out_shape=[jax.ShapeDtypeStruct((NP, D), jnp.float32),
               jax.ShapeDtypeStruct((NP, 1), jnp.float32)],
)


def _t23_body(p_ref, hp_ref, dinv_ref, b_ref, w_ref, out_ref):
    dinv = dinv_ref[...]
    h = jnp.maximum(dinv * (p_ref[0] + p_ref[1] + hp_ref[...]) + b_ref[...], 0.0)
    out_ref[...] = dinv * jnp.dot(h, w_ref[...],
                                  preferred_element_type=jnp.float32)


_t23 = pl.pallas_call(
    _t23_body,
    out_shape=jax.ShapeDtypeStruct((NP, D), jnp.float32),
)


def _t4_body(p_ref, hp_ref, dinv_ref, b_ref, batch_ref, wfc_ref, bfc_ref,
             out_ref, xmax_ref):
    dinv = dinv_ref[...]
    h = jnp.maximum(dinv * (p_ref[0] + p_ref[1] + hp_ref[...]) + b_ref[...], 0.0)
    bt = batch_ref[...].reshape(1, NP)
    gids = lax.broadcasted_iota(jnp.int32, (G, NP), 0)
    maskf = (gids == bt).astype(jnp.float32)
    sums = jnp.dot(maskf, h, preferred_element_type=jnp.float32)
    counts = jnp.sum(maskf, axis=1, keepdims=True)
    mean = sums / jnp.maximum(counts, 1.0)

    neg = jnp.float32(-jnp.inf)

    def body(g, carry):
        mg = batch_ref[...] == g
        v = jnp.where(mg, h, neg)
        xmax_ref[pl.ds(g, 1), :] = jnp.max(v, axis=0, keepdims=True)
        return carry

    lax.fori_loop(0, G, body, 0)
    xmax = jnp.where(counts > 0, xmax_ref[...], 0.0)
    pooled = jnp.concatenate([mean, xmax], axis=1)
    out_ref[...] = jnp.dot(pooled, wfc_ref[...],
                           preferred_element_type=jnp.float32) + bfc_ref[...]


_t4 = pl.pallas_call(
    _t4_body,
    out_shape=jax.ShapeDtypeStruct((G, CLS), jnp.float32),
    scratch_shapes=[pltpu.VMEM((G, D), jnp.float32)],
)


@jax.jit
def kernel(x, edge_index, batch, W1, b1, W2, b2, W3, b3, Wfc, bfc):
    ei = edge_index.astype(jnp.int32)
    pad = EP - E
    srcp = jnp.concatenate(
        [ei[0], jnp.full((pad,), N, jnp.int32)]).reshape(NC, NS, NCHUNK, CH)
    dstp = jnp.concatenate(
        [ei[1], jnp.full((pad,), N, jnp.int32)]).reshape(NC, NS, NCHUNK, CH)
    xp = jnp.pad(x, ((0, NP - N), (0, 0)))
    bt = jnp.pad(batch.astype(jnp.int32), (0, NP - N),
                 constant_values=G).reshape(NP, 1)
    zeros_deg = jnp.zeros((ROWS_PT, DW), jnp.float32)
    ones_deg = jnp.ones((CH, DW), jnp.float32)
    zeros_agg = jnp.zeros((CH, D), jnp.float32)

    degp = _deg(dstp, ones_deg, zeros_deg)
    hp1, dinv = _t1(degp, xp, W1)
    p1 = _agg(hp1, srcp, dstp, zeros_agg)
    hp2 = _t23(p1, hp1, dinv, b1.reshape(1, D), W2)
    p2 = _agg(hp2, srcp, dstp, zeros_agg)
    hp3 = _t23(p2, hp2, dinv, b2.reshape(1, D), W3)
    p3 = _agg(hp3, srcp, dstp, zeros_agg)
    return _t4(p3, hp3, dinv, b3.reshape(1, D), bt, Wfc, bfc.reshape(1, CLS))


# speedup vs baseline: 12.2165x; 1.5084x over previous
import jax
import jax.numpy as jnp
from jax import lax
from jax.experimental import pallas as pl
from jax.experimental.pallas import tpu as pltpu
from jax.experimental.pallas import tpu_sc as plsc

N = 10000
NP = 10240
D = 128
E = 320000
G = 64
CLS = 10
NC, NS = 2, 16
NW = NC * NS
CH = 128
NCHUNK = 80
EP = NW * NCHUNK * CH
ROWS_PT = NP // NS
ZCH = ROWS_PT // CH
DW = 8

_mesh = plsc.VectorSubcoreMesh(core_axis_name="c", subcore_axis_name="s")


def _deg_body(dsts_hbm, ones_hbm, zeros_hbm, out_hbm, acc, dst_cur, ones_v):
    c = lax.axis_index("c")
    s = lax.axis_index("s")
    base = s * ROWS_PT
    pltpu.sync_copy(zeros_hbm, acc.at[pl.ds(base, ROWS_PT)])
    pltpu.sync_copy(ones_hbm, ones_v)
    plsc.subcore_barrier()

    def body(j, carry):
        pltpu.sync_copy(dsts_hbm.at[c, s, j], dst_cur)
        pltpu.sync_copy(ones_v, acc.at[dst_cur], add=True)
        return carry

    lax.fori_loop(0, NCHUNK, body, 0)
    plsc.subcore_barrier()
    pltpu.sync_copy(acc.at[pl.ds(base, ROWS_PT)], out_hbm.at[c, pl.ds(base, ROWS_PT)])


_deg = pl.kernel(
    _deg_body,
    out_type=jax.ShapeDtypeStruct((NC, NP, DW), jnp.float32),
    mesh=_mesh,
    scratch_types=[
        pltpu.VMEM_SHARED((NP, DW), jnp.float32),
        pltpu.VMEM((CH,), jnp.int32),
        pltpu.VMEM((CH, DW), jnp.float32),
    ],
)


def _agg_body(hp_hbm, srcs_hbm, dsts_hbm, zeros_hbm, out_hbm,
              acc, src_cur, dst_cur, rows_v, sem):
    c = lax.axis_index("c")
    s = lax.axis_index("s")
    base = s * ROWS_PT
    for k in range(ZCH):
        pltpu.sync_copy(zeros_hbm, acc.at[pl.ds(base + k * CH, CH)])
    plsc.subcore_barrier()

    def body(j, carry):
        pltpu.sync_copy(srcs_hbm.at[c, s, j], src_cur)
        pltpu.sync_copy(dsts_hbm.at[c, s, j], dst_cur)
        pltpu.async_copy(hp_hbm.at[src_cur], rows_v, sem).wait()
        pltpu.sync_copy(rows_v, acc.at[dst_cur], add=True)
        return carry

    lax.fori_loop(0, NCHUNK, body, 0)
    plsc.subcore_barrier()
    pltpu.sync_copy(acc.at[pl.ds(base, ROWS_PT)], out_hbm.at[c, pl.ds(base, ROWS_PT)])


_agg = pl.kernel(
    _agg_body,
    out_type=jax.ShapeDtypeStruct((NC, NP, D), jnp.float32),
    mesh=_mesh,
    scratch_types=[
        pltpu.VMEM_SHARED((NP, D), jnp.float32),
        pltpu.VMEM((CH,), jnp.int32),
        pltpu.VMEM((CH,), jnp.int32),
        pltpu.VMEM((CH, D), jnp.float32),
        pltpu.SemaphoreType.DMA,
    ],
)


def _t1_body(degp_ref, x_ref, w_ref, hp_ref, dinv_ref):
    deg = degp_ref[0, :, 0:1] + degp_ref[1, :, 0:1]
    dinv = lax.rsqrt(deg + 1.0)
    dinv_ref[...] = dinv
    hp_ref[...] = dinv * jnp.dot(x_ref[...], w_ref[...],
                                 preferred_element_type=jnp.float32)


_t1 = pl.pallas_call(
    _t1_body,
    out_shape=[jax.ShapeDtypeStruct((NP, D), jnp.float32),
               jax.ShapeDtypeStruct((NP, 1), jnp.float32)],
)


def _t23_body(p_ref, hp_ref, dinv_ref, b_ref, w_ref, out_ref):
    dinv = dinv_ref[...]
    h = jnp.maximum(dinv * (p_ref[0] + p_ref[1] + hp_ref[...]) + b_ref[...], 0.0)
    out_ref[...] = dinv * jnp.dot(h, w_ref[...],
                                  preferred_element_type=jnp.float32)


_t23 = pl.pallas_call(
    _t23_body,
    out_shape=jax.ShapeDtypeStruct((NP, D), jnp.float32),
)


def _t4_body(p_ref, hp_ref, dinv_ref, b_ref, batch_ref, wfc_ref, bfc_ref,
             out_ref, xmax_ref):
    dinv = dinv_ref[...]
    h = jnp.maximum(dinv * (p_ref[0] + p_ref[1] + hp_ref[...]) + b_ref[...], 0.0)
    bt = batch_ref[...].reshape(1, NP)
    gids = lax.broadcasted_iota(jnp.int32, (G, NP), 0)
    maskf = (gids == bt).astype(jnp.float32)
    sums = jnp.dot(maskf, h, preferred_element_type=jnp.float32)
    counts = jnp.sum(maskf, axis=1, keepdims=True)
    mean = sums / jnp.maximum(counts, 1.0)

    neg = jnp.float32(-jnp.inf)

    def body(g, carry):
        mg = batch_ref[...] == g
        v = jnp.where(mg, h, neg)
        xmax_ref[pl.ds(g, 1), :] = jnp.max(v, axis=0, keepdims=True)
        return carry

    lax.fori_loop(0, G, body, 0)
    xmax = jnp.where(counts > 0, xmax_ref[...], 0.0)
    pooled = jnp.concatenate([mean, xmax], axis=1)
    out_ref[...] = jnp.dot(pooled, wfc_ref[...],
                           preferred_element_type=jnp.float32) + bfc_ref[...]


_t4 = pl.pallas_call(
    _t4_body,
    out_shape=jax.ShapeDtypeStruct((G, CLS), jnp.float32),
    scratch_shapes=[pltpu.VMEM((G, D), jnp.float32)],
)


@jax.jit
def kernel(x, edge_index, batch, W1, b1, W2, b2, W3, b3, Wfc, bfc):
    ei = edge_index.astype(jnp.int32)
    pad = EP - E
    pad_idx = N + jnp.arange(pad, dtype=jnp.int32) % (NP - N)
    srcp = jnp.concatenate([ei[0], pad_idx]).reshape(NC, NS, NCHUNK, CH)
    dstp = jnp.concatenate([ei[1], pad_idx]).reshape(NC, NS, NCHUNK, CH)
    xp = jnp.pad(x, ((0, NP - N), (0, 0)))
    bt = jnp.pad(batch.astype(jnp.int32), (0, NP - N),
                 constant_values=G).reshape(NP, 1)
    zeros_deg = jnp.zeros((ROWS_PT, DW), jnp.float32)
    ones_deg = jnp.ones((CH, DW), jnp.float32)
    zeros_agg = jnp.zeros((CH, D), jnp.float32)

    degp = _deg(dstp, ones_deg, zeros_deg)
    hp1, dinv = _t1(degp, xp, W1)
    p1 = _agg(hp1, srcp, dstp, zeros_agg)
    hp2 = _t23(p1, hp1, dinv, b1.reshape(1, D), W2)
    p2 = _agg(hp2, srcp, dstp, zeros_agg)
    hp3 = _t23(p2, hp2, dinv, b2.reshape(1, D), W3)
    p3 = _agg(hp3, srcp, dstp, zeros_agg)
    return _t4(p3, hp3, dinv, b3.reshape(1, D), bt, Wfc, bfc.reshape(1, CLS))
